# Initial kernel scaffold; baseline (speedup 1.0000x reference)
import jax
import jax.numpy as jnp
from jax.experimental import pallas as pl


def kernel(x_ids, degrees, edge_index, batch, emb, W1, a_src1, a_dst1, b1, W2, a_src2, a_dst2, b2, W3, a_src3, a_dst3, b3, Wl, bl):
    raise NotImplementedError("write your pallas kernel here")


# speedup vs baseline: 12.1421x; 12.1421x over previous
import functools

import jax
import jax.numpy as jnp
from jax import lax
from jax.experimental import pallas as pl
from jax.experimental.pallas import tpu as pltpu
from jax.experimental.pallas import tpu_sc as plsc

N = 50000
E = 800000
V = 100000
EMB = 62
G = 64
NCOUT = 16
C = 64
NPAD = 50176
RB = 1024
NBLK = NPAD // RB
W_PER = NPAD // 32
EB = 400
EPT = E // 16
LANES = 16
CHW = 16

_SC_PARAMS = pltpu.CompilerParams(
    use_tc_tiling_on_sc=False, needs_layout_passes=False
)


def _iota16():
    return lax.iota(jnp.int32, 16)


def _embed_body(ids_hbm, emb_hbm, x_hbm, idx_v, rows_v):
    wid = lax.axis_index("s") * 2 + lax.axis_index("c")
    base = wid * W_PER
    pltpu.sync_copy(ids_hbm.at[pl.ds(base, W_PER)], idx_v)
    pltpu.sync_copy(emb_hbm.at[idx_v], rows_v)
    pltpu.sync_copy(rows_v, x_hbm.at[pl.ds(base, W_PER)])


def _embed(ids, emb):
    f = pl.kernel(
        _embed_body,
        out_type=jax.ShapeDtypeStruct((NPAD, EMB), jnp.float32),
        mesh=plsc.VectorSubcoreMesh(core_axis_name="c", subcore_axis_name="s"),
        scratch_types=[
            pltpu.VMEM((W_PER,), jnp.int32),
            pltpu.VMEM((W_PER, EMB), jnp.float32),
        ],
        compiler_params=_SC_PARAMS,
    )
    return f(ids, emb)


def _edge_body(nch, heads, src_hbm, dst_hbm, al_hbm, hs_hbm, zz_hbm, zd_hbm,
               num_hbm, den_hbm, w_hbm,
               sidx, didx, asr, adr, wbuf, hrows, msg, acc_num, acc_den):
    c = lax.axis_index("c")
    s = lax.axis_index("s")
    zslice = pl.ds(s * (NPAD // 16), NPAD // 16)
    nblocks = EPT // EB

    pltpu.sync_copy(zd_hbm.at[pl.ds(0, EB)], wbuf)
    pltpu.sync_copy(zd_hbm.at[zslice], acc_den.at[zslice])
    plsc.subcore_barrier()

    def p1(k, _):
        base = s * EPT + k * EB
        pltpu.sync_copy(src_hbm.at[pl.ds(base, EB)], sidx)
        pltpu.sync_copy(dst_hbm.at[pl.ds(base, EB)], didx)
        pltpu.sync_copy(al_hbm.at[sidx], asr)
        pltpu.sync_copy(al_hbm.at[didx], adr)

        def grp(i, _):
            rows = i * LANES + _iota16()
            for h in range(heads):
                ch = jnp.full((16,), h, jnp.int32)
                e = (plsc.load_gather(asr, [rows, ch]) +
                     plsc.load_gather(adr, [rows, ch + 4]))
                e = jnp.where(e >= 0, e, 0.2 * e)
                plsc.store_scatter(wbuf, [rows, ch], jnp.exp(e))
            return 0
        lax.fori_loop(0, EB // LANES, grp, 0)
        pltpu.sync_copy(wbuf, w_hbm.at[c, pl.ds(base, EB)])
        pltpu.sync_copy(wbuf, acc_den.at[didx], add=True)
        return 0
    lax.fori_loop(0, nblocks, p1, 0)
    plsc.subcore_barrier()
    pltpu.sync_copy(acc_den.at[zslice], den_hbm.at[c, zslice])

    for cc in range(nch // 2):
        chunk = c * (nch // 2) + cc
        pltpu.sync_copy(zz_hbm.at[zslice], acc_num.at[zslice])
        plsc.subcore_barrier()

        def p2(k, _):
            base = s * EPT + k * EB
            pltpu.sync_copy(src_hbm.at[pl.ds(base, EB)], sidx)
            pltpu.sync_copy(dst_hbm.at[pl.ds(base, EB)], didx)
            pltpu.sync_copy(w_hbm.at[c, pl.ds(base, EB)], wbuf)

            def sh(i, _):
                sl = pl.ds(i * LANES, LANES)
                sidx[sl] = sidx[sl] + chunk * NPAD
                return 0
            lax.fori_loop(0, EB // LANES, sh, 0)
            pltpu.sync_copy(hs_hbm.at[sidx], hrows)

            hd = chunk // (nch // heads)
            hdv = jnp.full((16,), hd, jnp.int32)

            def grp(i, _):
                rows = i * LANES + _iota16()
                w16 = plsc.load_gather(wbuf, [rows, hdv])
                for j in range(CHW):
                    cj = jnp.full((16,), j, jnp.int32)
                    hv = plsc.load_gather(hrows, [rows, cj])
                    plsc.store_scatter(msg, [rows, cj], hv * w16)
                return 0
            lax.fori_loop(0, EB // LANES, grp, 0)
            pltpu.sync_copy(msg, acc_num.at[didx], add=True)
            return 0
        lax.fori_loop(0, nblocks, p2, 0)
        plsc.subcore_barrier()
        pltpu.sync_copy(acc_num.at[zslice],
                        num_hbm.at[zslice, pl.ds(chunk * CHW, CHW)])
        plsc.subcore_barrier()


def _edge(src, dst, al, hsflat, zz, zd, nch, heads):
    f = pl.kernel(
        functools.partial(_edge_body, nch, heads),
        out_type=(
            jax.ShapeDtypeStruct((NPAD, nch * CHW), jnp.float32),
            jax.ShapeDtypeStruct((2, NPAD, 4), jnp.float32),
            jax.ShapeDtypeStruct((2, E, 4), jnp.float32),
        ),
        mesh=plsc.VectorSubcoreMesh(core_axis_name="c", subcore_axis_name="s"),
        scratch_types=[
            pltpu.VMEM((EB,), jnp.int32),
            pltpu.VMEM((EB,), jnp.int32),
            pltpu.VMEM((EB, 8), jnp.float32),
            pltpu.VMEM((EB, 8), jnp.float32),
            pltpu.VMEM((EB, 4), jnp.float32),
            pltpu.VMEM((EB, CHW), jnp.float32),
            pltpu.VMEM((EB, CHW), jnp.float32),
            pltpu.VMEM_SHARED((NPAD, CHW), jnp.float32),
            pltpu.VMEM_SHARED((NPAD, 4), jnp.float32),
        ],
        compiler_params=_SC_PARAMS,
    )
    return f(src, dst, al, hsflat, zz, zd)


def _dense1_body(nch, x_ref, d_ref, wa_ref, wd_ref, ac_ref, hs_ref, al_ref):
    h = (jnp.dot(x_ref[...], wa_ref[...], preferred_element_type=jnp.float32)
         + jnp.dot(d_ref[...], wd_ref[...], preferred_element_type=jnp.float32))
    for cc in range(nch):
        hs_ref[cc] = h[:, cc * CHW:(cc + 1) * CHW]
    al_ref[...] = jnp.dot(h, ac_ref[...], preferred_element_type=jnp.float32)


def _dense1(xe, deg, wa, wd, ac, nch):
    return pl.pallas_call(
        functools.partial(_dense1_body, nch),
        grid=(NBLK,),
        in_specs=[
            pl.BlockSpec((RB, EMB), lambda i: (i, 0)),
            pl.BlockSpec((RB, 2), lambda i: (i, 0)),
            pl.BlockSpec((EMB, nch * CHW), lambda i: (0, 0)),
            pl.BlockSpec((2, nch * CHW), lambda i: (0, 0)),
            pl.BlockSpec((nch * CHW, 8), lambda i: (0, 0)),
        ],
        out_specs=[
            pl.BlockSpec((nch, RB, CHW), lambda i: (0, i, 0)),
            pl.BlockSpec((RB, 8), lambda i: (i, 0)),
        ],
        out_shape=[
            jax.ShapeDtypeStruct((nch, NPAD, CHW), jnp.float32),
            jax.ShapeDtypeStruct((NPAD, 8), jnp.float32),
        ],
    )(xe, deg, wa, wd, ac)


def _dense2_body(nch, heads_prev, num_ref, den_ref, b_ref, w_ref,
                 ac_ref, hs_ref, al_ref):
    num = num_ref[...]
    den = den_ref[0] + 1e-16
    cw = num.shape[1] // heads_prev
    parts = []
    for h in range(heads_prev):
        d = den[:, h:h + 1]
        parts.append(num[:, h * cw:(h + 1) * cw] / d)
    x = jnp.concatenate(parts, axis=1) if len(parts) > 1 else parts[0]
    x = jnp.maximum(x + b_ref[...], 0.0)
    h = jnp.dot(x, w_ref[...], preferred_element_type=jnp.float32)
    for cc in range(nch):
        hs_ref[cc] = h[:, cc * CHW:(cc + 1) * CHW]
    al_ref[...] = jnp.dot(h, ac_ref[...], preferred_element_type=jnp.float32)


def _dense2(num, den, b, w, ac, nch, heads_prev):
    dprev = num.shape[1]
    return pl.pallas_call(
        functools.partial(_dense2_body, nch, heads_prev),
        grid=(NBLK,),
        in_specs=[
            pl.BlockSpec((RB, dprev), lambda i: (i, 0)),
            pl.BlockSpec((2, RB, 4), lambda i: (0, i, 0)),
            pl.BlockSpec((1, dprev), lambda i: (0, 0)),
            pl.BlockSpec((dprev, nch * CHW), lambda i: (0, 0)),
            pl.BlockSpec((nch * CHW, 8), lambda i: (0, 0)),
        ],
        out_specs=[
            pl.BlockSpec((nch, RB, CHW), lambda i: (0, i, 0)),
            pl.BlockSpec((RB, 8), lambda i: (i, 0)),
        ],
        out_shape=[
            jax.ShapeDtypeStruct((nch, NPAD, CHW), jnp.float32),
            jax.ShapeDtypeStruct((NPAD, 8), jnp.float32),
        ],
    )(num, den, b.reshape(1, -1), w, ac)


def _final_body(num_ref, den_ref, b_ref, bt_ref, wl_ref, bl_ref, out_ref,
                sums, counts):
    i = pl.program_id(0)

    @pl.when(i == 0)
    def _():
        sums[...] = jnp.zeros_like(sums)
        counts[...] = jnp.zeros_like(counts)

    den = den_ref[0][:, 0:1] + 1e-16
    x = jnp.maximum(num_ref[...] / den + b_ref[...], 0.0)
    bt = bt_ref[...]
    oh = (bt == lax.broadcasted_iota(jnp.int32, (1, G), 1)).astype(jnp.float32)
    sums[...] += jnp.dot(oh.T, x, preferred_element_type=jnp.float32)
    counts[...] += jnp.sum(oh, axis=0, keepdims=True)

    @pl.when(i == NBLK - 1)
    def _():
        pooled = sums[...] / jnp.maximum(counts[...], 1.0).T
        out_ref[...] = (jnp.dot(pooled, wl_ref[...],
                                preferred_element_type=jnp.float32)
                        + bl_ref[...])


def _final(num3, den3, b3, batchp, wl, bl):
    return pl.pallas_call(
        _final_body,
        grid=(NBLK,),
        in_specs=[
            pl.BlockSpec((RB, C), lambda i: (i, 0)),
            pl.BlockSpec((2, RB, 4), lambda i: (0, i, 0)),
            pl.BlockSpec((1, C), lambda i: (0, 0)),
            pl.BlockSpec((RB, 1), lambda i: (i, 0)),
            pl.BlockSpec((C, NCOUT), lambda i: (0, 0)),
            pl.BlockSpec((1, NCOUT), lambda i: (0, 0)),
        ],
        out_specs=pl.BlockSpec((G, NCOUT), lambda i: (0, 0)),
        out_shape=jax.ShapeDtypeStruct((G, NCOUT), jnp.float32),
        scratch_shapes=[
            pltpu.VMEM((G, C), jnp.float32),
            pltpu.VMEM((1, G), jnp.float32),
        ],
    )(num3, den3, b3.reshape(1, -1), batchp, wl, bl.reshape(1, -1))


def _attn_mat(a_src, a_dst):
    h = a_src.shape[0]
    eye = jnp.eye(h, dtype=jnp.float32)
    asrc = (eye[:, None, :] * a_src[:, :, None]).reshape(h * C, h)
    adst = (eye[:, None, :] * a_dst[:, :, None]).reshape(h * C, h)
    z = jnp.zeros((h * C, 4 - h), jnp.float32)
    return jnp.concatenate([asrc, z, adst, z], axis=1)


def kernel(x_ids, degrees, edge_index, batch, emb, W1, a_src1, a_dst1, b1,
           W2, a_src2, a_dst2, b2, W3, a_src3, a_dst3, b3, Wl, bl):
    pad = NPAD - N
    ids = jnp.concatenate([x_ids.astype(jnp.int32), jnp.zeros((pad,), jnp.int32)])
    deg = jnp.concatenate([degrees, jnp.zeros((pad, 2), jnp.float32)])
    src = edge_index[0].astype(jnp.int32)
    dst = edge_index[1].astype(jnp.int32)
    batchp = jnp.concatenate([batch.astype(jnp.int32),
                              jnp.full((pad,), G, jnp.int32)]).reshape(NPAD, 1)
    zz = jnp.zeros((NPAD, CHW), jnp.float32)
    zd = jnp.zeros((NPAD, 4), jnp.float32)

    xe = _embed(ids, emb)

    hs, al = _dense1(xe, deg, W1[:EMB], W1[EMB:], _attn_mat(a_src1, a_dst1), 16)
    num, den, _ = _edge(src, dst, al, hs.reshape(16 * NPAD, CHW), zz, zd, 16, 4)
    hs, al = _dense2(num, den, b1, W2, _attn_mat(a_src2, a_dst2), 16, 4)
    num, den, _ = _edge(src, dst, al, hs.reshape(16 * NPAD, CHW), zz, zd, 16, 4)
    hs, al = _dense2(num, den, b2, W3, _attn_mat(a_src3, a_dst3), 4, 4)
    num, den, _ = _edge(src, dst, al, hs.reshape(4 * NPAD, CHW), zz, zd, 4, 1)

    return _final(num, den, b3, batchp, Wl, bl)
